# baseline (speedup 1.0000x reference)
import functools

import jax
import jax.numpy as jnp
from jax import lax
from jax.experimental import pallas as pl
from jax.experimental.pallas import tpu as pltpu
from jax.experimental.pallas import tpu_sc as plsc

N = 100000
D = 119
DP = 128
L = 16
NC, NS = 2, 16
NW = NC * NS
GROUPS = 16
BLOCK = GROUPS * L
BT = BLOCK // 8

NBLK = N // BLOCK
TFULL = NBLK // NW
XBLK = NBLK - TFULL * NW
TAIL0 = NBLK * BLOCK
TAILG = (N - TAIL0) // L

FIX_TILES = 125
FIX_ROWS = FIX_TILES * 8
FIX_GRID = N // FIX_ROWS


@functools.cache
def _build_sc():
    mesh = plsc.VectorSubcoreMesh(
        core_axis_name="c", subcore_axis_name="s", num_cores=NC, num_subcores=NS
    )

    @functools.partial(
        pl.kernel,
        out_type=jax.ShapeDtypeStruct((N, DP), jnp.int32),
        mesh=mesh,
        compiler_params=pltpu.CompilerParams(needs_layout_passes=False),
        scratch_types=[
            pltpu.VMEM((128,), jnp.int32),
            pltpu.VMEM((BLOCK,), jnp.int32),
            pltpu.VMEM((BT, 8, DP), jnp.int32),
        ],
    )
    def onehot_sc(z_hbm, table_hbm, out_hbm, table_v, z_v, buf):
        wid = lax.axis_index("s") * NC + lax.axis_index("c")
        iota = lax.iota(jnp.int32, L)
        ones = jnp.ones((L,), jnp.int32)
        zeros = jnp.zeros((L,), jnp.int32)
        sub = jnp.bitwise_and(iota, 7)
        tof = lax.shift_right_logical(iota, 3)

        pltpu.sync_copy(table_hbm, table_v)

        def _zero_tile(t, carry):
            for s in range(8):
                for g in range(8):
                    buf[t, s, pl.ds(g * L, L)] = zeros
            return carry

        lax.fori_loop(0, BT, _zero_tile, 0)

        def scatter_vals(ngroups, vals):
            for g in range(ngroups):
                zv = z_v[pl.ds(g * L, L)]
                idx = plsc.load_gather(table_v, [zv])
                plsc.store_scatter(buf, [2 * g + tof, sub, idx], vals)

        def emit(row0, ngroups):
            nrows = ngroups * L
            pltpu.sync_copy(z_hbm.at[pl.ds(row0, nrows)], z_v.at[pl.ds(0, nrows)])
            scatter_vals(ngroups, ones)
            pltpu.sync_copy(
                buf.at[pl.ds(0, nrows // 8)],
                out_hbm.reshape(N // 8, 8, DP).at[pl.ds(row0 // 8, nrows // 8)],
            )
            scatter_vals(ngroups, zeros)

        def round_body(t, carry):
            emit((t * NW + wid) * BLOCK, GROUPS)
            return carry

        lax.fori_loop(0, TFULL, round_body, 0)

        @pl.when(wid < XBLK)
        def _():
            emit((TFULL * NW + wid) * BLOCK, GROUPS)

        @pl.when(wid < TAILG)
        def _():
            emit(TAIL0 + wid * L, 1)

    return onehot_sc


def _fix_body(padded_ref, out_ref):
    out_ref[...] = padded_ref[:, :D]


@functools.cache
def _build_fix():
    return pl.pallas_call(
        _fix_body,
        grid=(FIX_GRID,),
        in_specs=[
            pl.BlockSpec((FIX_ROWS, DP), lambda i: (i, 0)),
        ],
        out_specs=pl.BlockSpec((FIX_ROWS, D), lambda i: (i, 0)),
        out_shape=jax.ShapeDtypeStruct((N, D), jnp.int32),
    )


def kernel(Z, z_to_index):
    zi = Z.astype(jnp.int32)
    table = jnp.zeros((128,), jnp.int32).at[:D].set(z_to_index.astype(jnp.int32))
    out3 = _build_sc()(zi, table)
    return _build_fix()(out3)


# speedup vs baseline: 1.8235x; 1.8235x over previous
import functools

import jax
import jax.numpy as jnp
from jax import lax
from jax.experimental import pallas as pl
from jax.experimental.pallas import tpu as pltpu
from jax.experimental.pallas import tpu_sc as plsc

N = 100000
D = 119
DP = 128
L = 16
NC, NS = 2, 16
NW = NC * NS
GROUPS = 16
BLOCK = GROUPS * L
BT = BLOCK // 8
ZRING = 4

NBLK = N // BLOCK
TFULL = NBLK // NW
XBLK = NBLK - TFULL * NW
KMAX = TFULL + 1
TAIL0 = NBLK * BLOCK
TAILG = (N - TAIL0) // L


@functools.cache
def _build_sc():
    mesh = plsc.VectorSubcoreMesh(
        core_axis_name="c", subcore_axis_name="s", num_cores=NC, num_subcores=NS
    )

    @functools.partial(
        pl.kernel,
        out_type=jax.ShapeDtypeStruct((N, DP), jnp.int32),
        mesh=mesh,
        compiler_params=pltpu.CompilerParams(needs_layout_passes=False),
        scratch_types=[
            pltpu.VMEM((128,), jnp.int32),
            pltpu.VMEM((ZRING, BLOCK), jnp.int32),
            pltpu.VMEM((2, BT, 8, DP), jnp.int32),
            pltpu.SemaphoreType.DMA,
            pltpu.SemaphoreType.DMA,
            pltpu.SemaphoreType.DMA,
            pltpu.SemaphoreType.DMA,
            pltpu.SemaphoreType.DMA,
            pltpu.SemaphoreType.DMA,
        ],
    )
    def onehot_sc(z_hbm, table_hbm, out_hbm, table_v, z_v, buf,
                  zs0, zs1, zs2, zs3, os0, os1):
        zsems = (zs0, zs1, zs2, zs3)
        osems = (os0, os1)
        out3 = out_hbm.reshape(N // 8, 8, DP)
        wid = lax.axis_index("s") * NC + lax.axis_index("c")
        iota = lax.iota(jnp.int32, L)
        ones = jnp.ones((L,), jnp.int32)
        zeros = jnp.zeros((L,), jnp.int32)
        sub = jnp.bitwise_and(iota, 7)
        tof = lax.shift_right_logical(iota, 3)
        in_x = wid < XBLK

        def row0_of(k):
            return (k * NW + wid) * BLOCK

        def start_z(k):
            pltpu.async_copy(
                z_hbm.at[pl.ds(row0_of(k), BLOCK)],
                z_v.at[k % ZRING],
                zsems[k % ZRING],
            )

        def wait_z(k):
            pltpu.make_async_copy(
                z_hbm.at[pl.ds(row0_of(k), BLOCK)],
                z_v.at[k % ZRING],
                zsems[k % ZRING],
            ).wait()

        def start_out(k):
            pltpu.async_copy(
                buf.at[k % 2],
                out3.at[pl.ds(row0_of(k) // 8, BT)],
                osems[k % 2],
            )

        def wait_out(k):
            pltpu.make_async_copy(
                buf.at[k % 2],
                out3.at[pl.ds(row0_of(k) // 8, BT)],
                osems[k % 2],
            ).wait()

        def scatter_vals(bank, slot, ngroups, vals):
            bvec = jnp.full((L,), bank, jnp.int32)
            for g in range(ngroups):
                zv = z_v[slot, pl.ds(g * L, L)]
                idx = plsc.load_gather(table_v, [zv])
                plsc.store_scatter(buf, [bvec, 2 * g + tof, sub, idx], vals)

        pltpu.sync_copy(table_hbm, table_v)

        def _zero_tile(t, carry):
            for b in range(2):
                for s in range(8):
                    for g in range(8):
                        buf[b, t, s, pl.ds(g * L, L)] = zeros
            return carry

        lax.fori_loop(0, BT, _zero_tile, 0)

        start_z(0)
        start_z(1)

        def step(k):
            if k >= 2:
                wait_out(k - 2)
                scatter_vals(k % 2, (k - 2) % ZRING, GROUPS, zeros)
            wait_z(k)
            scatter_vals(k % 2, k % ZRING, GROUPS, ones)
            start_out(k)
            if k + 2 < TFULL:
                start_z(k + 2)
            elif k + 2 == TFULL:
                @pl.when(in_x)
                def _():
                    start_z(k + 2)

        for k in range(KMAX):
            if k < TFULL:
                step(k)
            else:
                @pl.when(in_x)
                def _():
                    step(k)

        wait_out(TFULL - 2)
        wait_out(TFULL - 1)

        @pl.when(wid < TAILG)
        def _():
            trow = TAIL0 + wid * L
            for t in range(2):
                for s in range(8):
                    for g in range(8):
                        buf[0, t, s, pl.ds(g * L, L)] = zeros
            pltpu.sync_copy(
                z_hbm.at[pl.ds(trow, L)], z_v.at[0, pl.ds(0, L)]
            )
            zv = z_v[0, pl.ds(0, L)]
            idx = plsc.load_gather(table_v, [zv])
            plsc.store_scatter(
                buf, [jnp.zeros((L,), jnp.int32), tof, sub, idx], ones
            )
            pltpu.sync_copy(
                buf.at[0, pl.ds(0, 2)], out3.at[pl.ds(trow // 8, 2)]
            )

    return onehot_sc


def kernel(Z, z_to_index):
    zi = Z.astype(jnp.int32)
    table = jnp.zeros((128,), jnp.int32).at[:D].set(z_to_index.astype(jnp.int32))
    padded = _build_sc()(zi, table)
    return padded[:, :D]
